# baseline (speedup 1.0000x reference)
import jax
import jax.numpy as jnp
from jax import lax
from jax.experimental import pallas as pl
from jax.experimental.pallas import tpu as pltpu

N_NODES = 10000
N_EDGES = 320000
CH = 128

E = 512
NB = N_EDGES // E
G = 16
NG = E // G
W = 128
ACCN = 10240
NEG = -3.0e38


def _shift_up_f(a, k, fill):
    return jnp.concatenate(
        [a[k:], jnp.full((k,) + a.shape[1:], fill, a.dtype)], axis=0)


def _body(vid_ref, x_ref, w1t_ref, b1_ref, wet_ref, wvt_ref, out_ref, acc_ref):
    p = pl.program_id(0)
    b = pl.program_id(1)
    vcol = vid_ref[0]
    v_first = vcol[0, 0]
    v_last = vcol[E - 1, 0]
    base = (v_first // 8) * 8
    nchunks = (v_last - base) // W + 1
    vbb = jnp.broadcast_to(vcol, (E, CH))
    lane = lax.broadcasted_iota(jnp.int32, (E, W), 1)

    @pl.when((p == 0) & (b == 0))
    def _init():
        acc_ref[...] = jnp.full((ACCN, CH), NEG, jnp.float32)

    @pl.when(p == 0)
    def _phase0():
        xb = x_ref[...]
        z = jnp.dot(xb, w1t_ref[...], preferred_element_type=jnp.float32)
        z = z + b1_ref[...]
        s = z.astype(jnp.bfloat16)
        vb16 = vbb.astype(jnp.int16)
        k = 1
        while k < G:
            s = jnp.where(vb16 == _shift_up_f(vb16, k, -1),
                          jnp.maximum(s, _shift_up_f(s, k, NEG)), s)
            k *= 2
        heads = s.reshape(NG, G, CH)[:, 0, :]
        vheads = vb16.reshape(NG, G, CH)[:, 0, :]
        k = 1
        while k < NG:
            heads = jnp.where(vheads == _shift_up_f(vheads, k, -1),
                              jnp.maximum(heads, _shift_up_f(heads, k, NEG)),
                              heads)
            k *= 2
        tnext = _shift_up_f(heads, 1, NEG)
        vnext = _shift_up_f(vheads, 1, -1)
        trep = jnp.broadcast_to(tnext[:, None, :], (NG, G, CH)).reshape(E, CH)
        vrep = jnp.broadcast_to(vnext[:, None, :], (NG, G, CH)).reshape(E, CH)
        s = jnp.where(vb16 == vrep, jnp.maximum(s, trep), s)
        prevv = jnp.concatenate(
            [jnp.full((1, CH), -1, jnp.int16), vb16[:E - 1]], axis=0)
        firstb = (vb16 != prevv)
        lane16 = lane.astype(jnp.int16)
        one16 = jnp.bfloat16(1.0)
        zero16 = jnp.bfloat16(0.0)
        ones_ec = jnp.ones((E, CH), jnp.bfloat16)

        def chunk(j, _):
            start16 = (base + j * W).astype(jnp.int16)
            rel = vb16 - start16
            sel = jnp.where(firstb & (rel == lane16), one16, zero16)
            upd = lax.dot_general(sel, s, (((0,), (0,)), ((), ())),
                                  preferred_element_type=jnp.float32)
            cnt = lax.dot_general(sel, ones_ec, (((0,), (0,)), ((), ())),
                                  preferred_element_type=jnp.float32)
            cur = acc_ref[pl.ds(base + j * W, W), :]
            acc_ref[pl.ds(base + j * W, W), :] = jnp.where(
                cnt > 0.0, jnp.maximum(cur, upd), cur)
            return 0

        lax.fori_loop(0, nchunks, chunk, 0)

    @pl.when((p == 1) & (b == 0))
    def _apply_wv():
        def step(i, _):
            sl = acc_ref[pl.ds(i * 512, 512), :]
            sl = jnp.where(sl == NEG, 0.0, sl)
            acc_ref[pl.ds(i * 512, 512), :] = jnp.dot(
                sl, wvt_ref[...], preferred_element_type=jnp.float32)
            return 0
        lax.fori_loop(0, ACCN // 512, step, 0)

    @pl.when(p == 1)
    def _phase1():
        xb = x_ref[...]
        ze = jnp.dot(xb, wet_ref[...], preferred_element_type=jnp.float32)

        def chunk(j, g):
            start = base + j * W
            rel = vbb - start
            sel = jnp.where(rel == lane, 1.0, 0.0)
            zw = acc_ref[pl.ds(start, W), :]
            return g + jnp.dot(sel, zw, preferred_element_type=jnp.float32)

        g = lax.fori_loop(0, nchunks, chunk, jnp.zeros((E, CH), jnp.float32))
        out_ref[...] = ze + g


def kernel(x, vertex_id, W1, b1, We, Wv):
    vid3 = vertex_id.astype(jnp.int32).reshape(NB, E, 1)
    grid = (2, NB)
    return pl.pallas_call(
        _body,
        grid=grid,
        in_specs=[
            pl.BlockSpec((1, E, 1), lambda p, b: (b, 0, 0)),
            pl.BlockSpec((E, CH), lambda p, b: (b, 0)),
            pl.BlockSpec((CH, CH), lambda p, b: (0, 0)),
            pl.BlockSpec((1, CH), lambda p, b: (0, 0)),
            pl.BlockSpec((CH, CH), lambda p, b: (0, 0)),
            pl.BlockSpec((CH, CH), lambda p, b: (0, 0)),
        ],
        out_specs=pl.BlockSpec(
            (E, CH), lambda p, b: (jnp.where(p == 1, b, 0), 0)),
        out_shape=jax.ShapeDtypeStruct((N_EDGES, CH), jnp.float32),
        scratch_shapes=[pltpu.VMEM((ACCN, CH), jnp.float32)],
        compiler_params=pltpu.CompilerParams(
            dimension_semantics=("arbitrary", "arbitrary")),
    )(vid3, x, W1.T, b1.reshape(1, CH), We.T, Wv.T)


# speedup vs baseline: 1.4573x; 1.4573x over previous
import jax
import jax.numpy as jnp
from jax import lax
from jax.experimental import pallas as pl
from jax.experimental.pallas import tpu as pltpu

N_NODES = 10000
N_EDGES = 320000
CH = 128

E = 2560
NB = N_EDGES // E
G = 16
NG = E // G
W = 128
ACCN = 10240
NEG = -3.0e38


def _shift_up_f(a, k, fill):
    return jnp.concatenate(
        [a[k:], jnp.full((k,) + a.shape[1:], fill, a.dtype)], axis=0)


def _body(vid_ref, x_ref, w1t_ref, b1_ref, wet_ref, wvt_ref, out_ref, acc_ref):
    p = pl.program_id(0)
    b = pl.program_id(1)
    vcol = vid_ref[0]
    v_first = vcol[0, 0]
    v_last = vcol[E - 1, 0]
    base = (v_first // 8) * 8
    nchunks = (v_last - base) // W + 1
    vbb = jnp.broadcast_to(vcol, (E, CH))
    lane = lax.broadcasted_iota(jnp.int32, (E, W), 1)

    @pl.when((p == 0) & (b == 0))
    def _init():
        acc_ref[...] = jnp.full((ACCN, CH), NEG, jnp.float32)

    @pl.when(p == 0)
    def _phase0():
        xb = x_ref[...].astype(jnp.bfloat16)
        z = jnp.dot(xb, w1t_ref[...], preferred_element_type=jnp.float32)
        z = z + b1_ref[...]
        s = z.astype(jnp.bfloat16)
        vb16 = vbb.astype(jnp.int16)
        k = 1
        while k < G:
            s = jnp.where(vb16 == _shift_up_f(vb16, k, -1),
                          jnp.maximum(s, _shift_up_f(s, k, NEG)), s)
            k *= 2
        heads = s.reshape(NG, G, CH)[:, 0, :]
        vheads = vb16.reshape(NG, G, CH)[:, 0, :]
        k = 1
        while k < NG:
            heads = jnp.where(vheads == _shift_up_f(vheads, k, -1),
                              jnp.maximum(heads, _shift_up_f(heads, k, NEG)),
                              heads)
            k *= 2
        tnext = _shift_up_f(heads, 1, NEG)
        vnext = _shift_up_f(vheads, 1, -1)
        trep = jnp.broadcast_to(tnext[:, None, :], (NG, G, CH)).reshape(E, CH)
        vrep = jnp.broadcast_to(vnext[:, None, :], (NG, G, CH)).reshape(E, CH)
        s = jnp.where(vb16 == vrep, jnp.maximum(s, trep), s)
        prevv = jnp.concatenate(
            [jnp.full((1, CH), -1, jnp.int16), vb16[:E - 1]], axis=0)
        firstb = (vb16 != prevv)
        lane16 = lane.astype(jnp.int16)
        one16 = jnp.bfloat16(1.0)
        zero16 = jnp.bfloat16(0.0)
        ones_ec = jnp.ones((E, CH), jnp.bfloat16)

        def chunk(j, _):
            start16 = (base + j * W).astype(jnp.int16)
            rel = vb16 - start16
            sel = jnp.where(firstb & (rel == lane16), one16, zero16)
            upd = lax.dot_general(sel, s, (((0,), (0,)), ((), ())),
                                  preferred_element_type=jnp.float32)
            cnt = lax.dot_general(sel, ones_ec, (((0,), (0,)), ((), ())),
                                  preferred_element_type=jnp.float32)
            cur = acc_ref[pl.ds(base + j * W, W), :]
            acc_ref[pl.ds(base + j * W, W), :] = jnp.where(
                cnt > 0.0, jnp.maximum(cur, upd), cur)
            return 0

        lax.fori_loop(0, nchunks, chunk, 0)

    @pl.when((p == 1) & (b == 0))
    def _apply_wv():
        def step(i, _):
            sl = acc_ref[pl.ds(i * 512, 512), :]
            sl = jnp.where(sl == NEG, 0.0, sl)
            acc_ref[pl.ds(i * 512, 512), :] = jnp.dot(
                sl, wvt_ref[...], preferred_element_type=jnp.float32)
            return 0
        lax.fori_loop(0, ACCN // 512, step, 0)

    @pl.when(p == 1)
    def _phase1():
        xb = x_ref[...].astype(jnp.bfloat16)
        ze = jnp.dot(xb, wet_ref[...], preferred_element_type=jnp.float32)
        vb16 = vbb.astype(jnp.int16)
        lane16 = lane.astype(jnp.int16)

        def chunk(j, g):
            rel = vb16 - (base + j * W).astype(jnp.int16)
            sel = jnp.where(rel == lane16,
                            jnp.bfloat16(1.0), jnp.bfloat16(0.0))
            zw = acc_ref[pl.ds(base + j * W, W), :].astype(jnp.bfloat16)
            return g + jnp.dot(sel, zw, preferred_element_type=jnp.float32)

        g = lax.fori_loop(0, nchunks, chunk, jnp.zeros((E, CH), jnp.float32))
        out_ref[...] = ze + g


def kernel(x, vertex_id, W1, b1, We, Wv):
    vid3 = vertex_id.astype(jnp.int32).reshape(NB, E, 1)
    grid = (2, NB)
    return pl.pallas_call(
        _body,
        grid=grid,
        in_specs=[
            pl.BlockSpec((1, E, 1), lambda p, b: (b, 0, 0)),
            pl.BlockSpec((E, CH), lambda p, b: (b, 0)),
            pl.BlockSpec((CH, CH), lambda p, b: (0, 0)),
            pl.BlockSpec((1, CH), lambda p, b: (0, 0)),
            pl.BlockSpec((CH, CH), lambda p, b: (0, 0)),
            pl.BlockSpec((CH, CH), lambda p, b: (0, 0)),
        ],
        out_specs=pl.BlockSpec(
            (E, CH), lambda p, b: (jnp.where(p == 1, b, 0), 0)),
        out_shape=jax.ShapeDtypeStruct((N_EDGES, CH), jnp.float32),
        scratch_shapes=[pltpu.VMEM((ACCN, CH), jnp.float32)],
        compiler_params=pltpu.CompilerParams(
            dimension_semantics=("arbitrary", "arbitrary")),
    )(vid3, x, W1.T.astype(jnp.bfloat16), b1.reshape(1, CH),
      We.T.astype(jnp.bfloat16), Wv.T)
